# baseline (speedup 1.0000x reference)
import functools

import jax
import jax.numpy as jnp
from jax import lax
from jax.experimental import pallas as pl
from jax.experimental.pallas import tpu as pltpu
from jax.experimental.pallas import tpu_sc as plsc

_N = 300
_NPAD = 304
_LANES = 16
_NWIN = _NPAD // _LANES

_NW = 32
_CH = 40


def _rbf_sc(d_hbm, c_hbm, out_hbm, d_v, c_v, ob, sems):
    rows_pw = d_v.shape[0] - _LANES
    nch = rows_pw // _CH
    wid = lax.axis_index("s") * 2 + lax.axis_index("c")
    base = wid * rows_pw

    pltpu.sync_copy(d_hbm.at[pl.ds(base, rows_pw)], d_v.at[pl.ds(0, rows_pw)])
    pltpu.sync_copy(c_hbm, c_v)

    c_head = c_v[pl.ds(0, _LANES)]
    gap_v = jnp.broadcast_to(c_head[1] - c_head[0], (_LANES,))
    coef = -1.0 / gap_v

    lane_iota = lax.iota(jnp.int32, _LANES)
    tail0 = (_NWIN - 1) * _LANES
    tail_cols = tail0 + lane_iota
    tail_mask = tail_cols < _N
    c_tail = c_v[pl.ds(tail0, _LANES)]
    zero_v = jnp.zeros((_LANES,), jnp.float32)
    c0_v = jnp.broadcast_to(c_head[0], (_LANES,))
    invgap_v = 1.0 / gap_v

    def chunk_body(k, carry):
        b = lax.rem(k, 4)

        @pl.when(k >= 4)
        def _():
            pltpu.make_async_copy(
                ob.at[b], out_hbm.at[pl.ds(base + (k - 4) * _CH, _CH)],
                sems.at[b]).wait()

        def row8_body(q, carry2):
            dvec = d_v[pl.ds(k * _CH + q * 8, _LANES)]
            jc_i = ((dvec - c0_v) * invgap_v).astype(jnp.int32)
            w0_v = jnp.clip((jc_i - 34) & (-_LANES), 0, 192)
            for rr in range(8):
                r = q * 8 + rr
                d_s = dvec[rr]
                for w in range(_NWIN - 1):
                    ob[b, r, pl.ds(w * _LANES, _LANES)] = zero_v
                w0 = pl.multiple_of(w0_v[rr], _LANES)
                for i in range(6):
                    off = w0 + i * _LANES
                    c = c_v[pl.ds(off, _LANES)]
                    x = d_s - c
                    ob[b, r, pl.ds(off, _LANES)] = jnp.exp(coef * (x * x))
                xt = d_s - c_tail
                vt = jnp.exp(coef * (xt * xt))
                plsc.store_scatter(
                    ob,
                    [jnp.broadcast_to(b, (_LANES,)),
                     jnp.broadcast_to(r, (_LANES,)),
                     tail_cols],
                    vt, mask=tail_mask)
            return carry2

        lax.fori_loop(0, _CH // 8, row8_body, 0)

        pltpu.make_async_copy(
            ob.at[b], out_hbm.at[pl.ds(base + k * _CH, _CH)],
            sems.at[b]).start()
        return carry

    lax.fori_loop(0, nch, chunk_body, 0)

    for b2 in range(4):
        k = nch - 4 + b2
        b = k % 4
        pltpu.make_async_copy(
            ob.at[b], out_hbm.at[pl.ds(base + k * _CH, _CH)],
            sems.at[b]).wait()


def kernel(distance, centers):
    E = distance.shape[0]
    n = centers.shape[0]
    rows_pw = E // _NW
    d1 = distance.reshape(E)
    cpad = jnp.pad(centers, (0, _NPAD - n), constant_values=1e4)

    mesh = plsc.VectorSubcoreMesh(core_axis_name="c", subcore_axis_name="s")
    f = functools.partial(
        pl.kernel,
        mesh=mesh,
        out_type=jax.ShapeDtypeStruct((E, n), jnp.float32),
        scratch_types=[
            pltpu.VMEM((rows_pw + _LANES,), jnp.float32),
            pltpu.VMEM((_NPAD,), jnp.float32),
            pltpu.VMEM((4, _CH, n), jnp.float32),
            pltpu.SemaphoreType.DMA((4,)),
        ],
        compiler_params=pltpu.CompilerParams(
            needs_layout_passes=False),
    )(_rbf_sc)
    return f(d1, cpad)


# speedup vs baseline: 1.3560x; 1.3560x over previous
import functools

import jax
import jax.numpy as jnp
from jax import lax
from jax.experimental import pallas as pl
from jax.experimental.pallas import tpu as pltpu
from jax.experimental.pallas import tpu_sc as plsc

_N = 300
_NPAD = 304
_LANES = 16
_NWIN = _NPAD // _LANES

_NW = 32
_CH = 40


def _rbf_sc(d_hbm, c_hbm, out_hbm, d_v, c_v, ob, sems):
    rows_pw = d_v.shape[0] - _LANES
    nch = rows_pw // _CH
    wid = lax.axis_index("s") * 2 + lax.axis_index("c")
    base = wid * rows_pw

    pltpu.sync_copy(d_hbm.at[pl.ds(base, rows_pw)], d_v.at[pl.ds(0, rows_pw)])
    pltpu.sync_copy(c_hbm, c_v)

    c_head = c_v[pl.ds(0, _LANES)]
    gap_v = jnp.broadcast_to(c_head[1] - c_head[0], (_LANES,))
    coef = -1.0 / gap_v

    lane_iota = lax.iota(jnp.int32, _LANES)
    tail0 = (_NWIN - 1) * _LANES
    tail_cols = tail0 + lane_iota
    tail_mask = tail_cols < _N
    cs = [c_v[pl.ds(w * _LANES, _LANES)] for w in range(_NWIN)]

    def chunk_body(k, carry):
        b = lax.rem(k, 4)

        @pl.when(k >= 4)
        def _():
            pltpu.make_async_copy(
                ob.at[b], out_hbm.at[pl.ds(base + (k - 4) * _CH, _CH)],
                sems.at[b]).wait()

        def row8_body(q, carry2):
            dvec = d_v[pl.ds(k * _CH + q * 8, _LANES)]
            for rr in range(8):
                r = q * 8 + rr
                d_s = dvec[rr]
                for w in range(_NWIN - 1):
                    x = d_s - cs[w]
                    ob[b, r, pl.ds(w * _LANES, _LANES)] = jnp.exp(coef * (x * x))
                xt = d_s - cs[_NWIN - 1]
                vt = jnp.exp(coef * (xt * xt))
                plsc.store_scatter(
                    ob,
                    [jnp.broadcast_to(b, (_LANES,)),
                     jnp.broadcast_to(r, (_LANES,)),
                     tail_cols],
                    vt, mask=tail_mask)
            return carry2

        for q in range(_CH // 8):
            row8_body(q, 0)

        pltpu.make_async_copy(
            ob.at[b], out_hbm.at[pl.ds(base + k * _CH, _CH)],
            sems.at[b]).start()
        return carry

    lax.fori_loop(0, nch, chunk_body, 0)

    for b2 in range(4):
        k = nch - 4 + b2
        b = k % 4
        pltpu.make_async_copy(
            ob.at[b], out_hbm.at[pl.ds(base + k * _CH, _CH)],
            sems.at[b]).wait()


def kernel(distance, centers):
    E = distance.shape[0]
    n = centers.shape[0]
    rows_pw = E // _NW
    d1 = distance.reshape(E)
    cpad = jnp.pad(centers, (0, _NPAD - n), constant_values=1e4)

    mesh = plsc.VectorSubcoreMesh(core_axis_name="c", subcore_axis_name="s")
    f = functools.partial(
        pl.kernel,
        mesh=mesh,
        out_type=jax.ShapeDtypeStruct((E, n), jnp.float32),
        scratch_types=[
            pltpu.VMEM((rows_pw + _LANES,), jnp.float32),
            pltpu.VMEM((_NPAD,), jnp.float32),
            pltpu.VMEM((4, _CH, n), jnp.float32),
            pltpu.SemaphoreType.DMA((4,)),
        ],
        compiler_params=pltpu.CompilerParams(
            needs_layout_passes=False),
    )(_rbf_sc)
    return f(d1, cpad)


# speedup vs baseline: 1.6810x; 1.2397x over previous
import functools

import jax
import jax.numpy as jnp
from jax import lax
from jax.experimental import pallas as pl
from jax.experimental.pallas import tpu as pltpu
from jax.experimental.pallas import tpu_sc as plsc

_N = 300
_NPAD = 304
_LANES = 16
_NWIN = _NPAD // _LANES

_NW = 32
_CH = 40


def _rbf_sc(d_hbm, c_hbm, out_hbm, d_v, c_v, ob, sems):
    rows_pw = d_v.shape[0] - _LANES
    nch = rows_pw // _CH
    wid = lax.axis_index("s") * 2 + lax.axis_index("c")
    base = wid * rows_pw

    pltpu.sync_copy(d_hbm.at[pl.ds(base, rows_pw)], d_v.at[pl.ds(0, rows_pw)])
    pltpu.sync_copy(c_hbm, c_v)

    c_head = c_v[pl.ds(0, _LANES)]
    gap_v = jnp.broadcast_to(c_head[1] - c_head[0], (_LANES,))
    coef = -1.0 / gap_v

    lane_iota = lax.iota(jnp.int32, _LANES)
    tail0 = (_NWIN - 1) * _LANES
    tail_cols = tail0 + lane_iota
    tail_mask = tail_cols < _N
    cs = [c_v[pl.ds(w * _LANES, _LANES)] for w in range(_NWIN)]

    def chunk_body(k, carry):
        b = lax.rem(k, 4)

        @pl.when(k >= 4)
        def _():
            pltpu.make_async_copy(
                ob.at[b], out_hbm.at[pl.ds(base + (k - 4) * _CH, _CH)],
                sems.at[b]).wait()

        def row8_body(q, carry2):
            dvec = d_v[pl.ds(k * _CH + q * 8, _LANES)]
            for rr in range(8):
                r = q * 8 + rr
                d_s = dvec[rr]
                for w in range(_NWIN - 1):
                    x = d_s - cs[w]
                    ob[b, r, pl.ds(w * _LANES, _LANES)] = jnp.exp(coef * (x * x))
                xt = d_s - cs[_NWIN - 1]
                vt = jnp.exp(coef * (xt * xt))
                plsc.store_scatter(
                    ob,
                    [jnp.broadcast_to(b, (_LANES,)),
                     jnp.broadcast_to(r, (_LANES,)),
                     tail_cols],
                    vt, mask=tail_mask)
            return carry2

        lax.fori_loop(0, _CH // 8, row8_body, 0)

        pltpu.make_async_copy(
            ob.at[b], out_hbm.at[pl.ds(base + k * _CH, _CH)],
            sems.at[b]).start()
        return carry

    lax.fori_loop(0, nch, chunk_body, 0)

    for b2 in range(4):
        k = nch - 4 + b2
        b = k % 4
        pltpu.make_async_copy(
            ob.at[b], out_hbm.at[pl.ds(base + k * _CH, _CH)],
            sems.at[b]).wait()


def kernel(distance, centers):
    E = distance.shape[0]
    n = centers.shape[0]
    rows_pw = E // _NW
    d1 = distance.reshape(E)
    cpad = jnp.pad(centers, (0, _NPAD - n), constant_values=1e4)

    mesh = plsc.VectorSubcoreMesh(core_axis_name="c", subcore_axis_name="s")
    f = functools.partial(
        pl.kernel,
        mesh=mesh,
        out_type=jax.ShapeDtypeStruct((E, n), jnp.float32),
        scratch_types=[
            pltpu.VMEM((rows_pw + _LANES,), jnp.float32),
            pltpu.VMEM((_NPAD,), jnp.float32),
            pltpu.VMEM((4, _CH, n), jnp.float32),
            pltpu.SemaphoreType.DMA((4,)),
        ],
        compiler_params=pltpu.CompilerParams(
            needs_layout_passes=False),
    )(_rbf_sc)
    return f(d1, cpad)
